# Initial kernel scaffold; baseline (speedup 1.0000x reference)
import jax
import jax.numpy as jnp
from jax.experimental import pallas as pl


def kernel(inputs, codebook):
    raise NotImplementedError("write your pallas kernel here")


# speedup vs baseline: 1.2148x; 1.2148x over previous
import functools

import jax
import jax.numpy as jnp
from jax import lax
from jax.experimental import pallas as pl
from jax.experimental.pallas import tpu as pltpu
from jax.experimental.pallas import tpu_sc as plsc

BT = 16384
D = 256
K = 8192
MT = 256
CK = 2048
NK = K // CK

NC = 2
NS = 16
NW = NC * NS
ROWS_W = BT // NW
CH = 128
NCH = ROWS_W // CH


PHASES = ((0, 2816), (2816, 5632), (5632, 8192))


def _assign_body(x_ref, xsq_ref, cb_ref, csq_ref, out_ref):
    x = x_ref[...]
    xsq = xsq_ref[...]
    x16 = x.astype(jnp.bfloat16)
    acc_v = None
    acc_i = None
    for (lo, hi) in PHASES:
        cb = cb_ref[lo:hi, :]
        csq = csq_ref[:, lo:hi]
        cross = lax.dot_general(x16, cb.astype(jnp.bfloat16),
                                (((1,), (1,)), ((), ())),
                                preferred_element_type=jnp.float32)
        s = jnp.maximum((xsq + csq) - 2.0 * cross, 0.0)
        d = jnp.where(s == 0.0, 0.0, s * lax.rsqrt(s))
        lv = jnp.min(d, axis=1, keepdims=True)
        iota = lax.broadcasted_iota(jnp.int32, d.shape, 1) + lo
        li = jnp.min(jnp.where(d == lv, iota, K), axis=1, keepdims=True)
        if acc_v is None:
            acc_v, acc_i = lv, li
        else:
            better = (lv < acc_v) | ((lv == acc_v) & (li < acc_i))
            acc_i = jnp.where(better, li, acc_i)
            acc_v = jnp.where(better, lv, acc_v)
        u = lax.bitcast_convert_type(acc_v, jnp.uint32)
        u = (u + jnp.uint32(0x7FFF) + ((u >> 16) & jnp.uint32(1))) & jnp.uint32(0xFFFF0000)
        acc_v = lax.bitcast_convert_type(u, jnp.float32)
    out_ref[...] = acc_i


def _assign(x, xsq, codebook, csq):
    return pl.pallas_call(
        _assign_body,
        grid=(BT // MT,),
        in_specs=[
            pl.BlockSpec((MT, D), lambda m: (m, 0)),
            pl.BlockSpec((MT, 1), lambda m: (m, 0)),
            pl.BlockSpec((K, D), lambda m: (0, 0)),
            pl.BlockSpec((1, K), lambda m: (0, 0)),
        ],
        out_specs=pl.BlockSpec((MT, 1), lambda m: (m, 0)),
        out_shape=jax.ShapeDtypeStruct((BT, 1), jnp.int32),
    )(x, xsq, codebook, csq)


def _gather_body(idx_hbm, cb_hbm, out_hbm, idx_v, buf0, buf1, s0, s1):
    wid = lax.axis_index("s") * NC + lax.axis_index("c")
    base = wid * ROWS_W
    pltpu.sync_copy(idx_hbm.at[wid], idx_v)
    bufs = (buf0, buf1)
    sems = (s0, s1)
    cp = pltpu.async_copy(cb_hbm.at[idx_v.at[0]], bufs[0], sems[0])
    for c in range(NCH):
        cp.wait()
        if c + 1 < NCH:
            cp = pltpu.async_copy(
                cb_hbm.at[idx_v.at[c + 1]], bufs[(c + 1) % 2], sems[(c + 1) % 2])
        pltpu.sync_copy(bufs[c % 2], out_hbm.at[pl.ds(base + c * CH, CH)])


def _gather(idx, codebook):
    mesh = plsc.VectorSubcoreMesh(core_axis_name="c", subcore_axis_name="s")
    f = pl.kernel(
        _gather_body,
        out_type=jax.ShapeDtypeStruct((BT, D), jnp.float32),
        mesh=mesh,
        scratch_types=[
            pltpu.VMEM((NCH, CH), jnp.int32),
            pltpu.VMEM((CH, D), jnp.float32),
            pltpu.VMEM((CH, D), jnp.float32),
            pltpu.SemaphoreType.DMA,
            pltpu.SemaphoreType.DMA,
        ],
    )
    return f(idx, codebook)


def kernel(inputs, codebook):
    b, t, d = inputs.shape
    x = inputs.reshape(b * t, d)
    csq = lax.optimization_barrier(
        jnp.sum(codebook * codebook, axis=-1))[None, :]
    xsq = lax.optimization_barrier(
        jnp.sum(x * x, axis=-1, keepdims=True))
    idx = _assign(x, xsq, codebook, csq)
    idx = idx.reshape(NW, NCH, CH)
    quant = _gather(idx, codebook)
    return quant.reshape(b, t, d)


# speedup vs baseline: 1.3921x; 1.1459x over previous
import functools

import jax
import jax.numpy as jnp
from jax import lax
from jax.experimental import pallas as pl
from jax.experimental.pallas import tpu as pltpu
from jax.experimental.pallas import tpu_sc as plsc

BT = 16384
D = 256
K = 8192
MT = 256
CK = 2048
NK = K // CK

NC = 2
NS = 16
NW = NC * NS
ROWS_W = BT // NW
CH = 128
NCH = ROWS_W // CH


PHASES = ((0, 2816), (2816, 5632), (5632, 8192))


def _assign_body(x_ref, xsq_ref, cb_ref, csq_ref, out_ref):
    x = x_ref[...]
    xsq = xsq_ref[...]
    x2_16 = (2.0 * x).astype(jnp.bfloat16)
    lane = lax.broadcasted_iota(jnp.int32, (MT, 128), 1)
    acc_v = None
    acc_i = None
    for (lo, hi) in PHASES:
        cb = cb_ref[lo:hi, :]
        csq = csq_ref[:, lo:hi]
        cross2 = lax.dot_general(x2_16, cb.astype(jnp.bfloat16),
                                 (((1,), (1,)), ((), ())),
                                 preferred_element_type=jnp.float32)
        s = jnp.maximum((xsq + csq) - cross2, 0.0)
        d = jnp.where(s == 0.0, 0.0, s * lax.rsqrt(s))
        pv = d[:, 0:128]
        pj = jnp.zeros((MT, 128), dtype=jnp.int32)
        for j in range(1, (hi - lo) // 128):
            dj = d[:, j * 128:(j + 1) * 128]
            better = dj < pv
            pv = jnp.where(better, dj, pv)
            pj = jnp.where(better, j, pj)
        pk = lo + pj * 128 + lane
        lv = jnp.min(pv, axis=1, keepdims=True)
        li = jnp.min(jnp.where(pv == lv, pk, K), axis=1, keepdims=True)
        if acc_v is None:
            acc_v, acc_i = lv, li
        else:
            better = (lv < acc_v) | ((lv == acc_v) & (li < acc_i))
            acc_i = jnp.where(better, li, acc_i)
            acc_v = jnp.where(better, lv, acc_v)
        u = lax.bitcast_convert_type(acc_v, jnp.uint32)
        u = (u + jnp.uint32(0x7FFF) + ((u >> 16) & jnp.uint32(1))) & jnp.uint32(0xFFFF0000)
        acc_v = lax.bitcast_convert_type(u, jnp.float32)
    out_ref[...] = acc_i


def _assign(x, xsq, codebook, csq):
    return pl.pallas_call(
        _assign_body,
        grid=(BT // MT,),
        in_specs=[
            pl.BlockSpec((MT, D), lambda m: (m, 0)),
            pl.BlockSpec((MT, 1), lambda m: (m, 0)),
            pl.BlockSpec((K, D), lambda m: (0, 0)),
            pl.BlockSpec((1, K), lambda m: (0, 0)),
        ],
        out_specs=pl.BlockSpec((MT, 1), lambda m: (m, 0)),
        out_shape=jax.ShapeDtypeStruct((BT, 1), jnp.int32),
    )(x, xsq, codebook, csq)


def _gather_body(idx_hbm, cb_hbm, out_hbm, idx_v, buf0, buf1, s0, s1):
    wid = lax.axis_index("s") * NC + lax.axis_index("c")
    base = wid * ROWS_W
    pltpu.sync_copy(idx_hbm.at[wid], idx_v)
    bufs = (buf0, buf1)
    sems = (s0, s1)
    cp = pltpu.async_copy(cb_hbm.at[idx_v.at[0]], bufs[0], sems[0])
    for c in range(NCH):
        cp.wait()
        if c + 1 < NCH:
            cp = pltpu.async_copy(
                cb_hbm.at[idx_v.at[c + 1]], bufs[(c + 1) % 2], sems[(c + 1) % 2])
        pltpu.sync_copy(bufs[c % 2], out_hbm.at[pl.ds(base + c * CH, CH)])


def _gather(idx, codebook):
    mesh = plsc.VectorSubcoreMesh(core_axis_name="c", subcore_axis_name="s")
    f = pl.kernel(
        _gather_body,
        out_type=jax.ShapeDtypeStruct((BT, D), jnp.float32),
        mesh=mesh,
        scratch_types=[
            pltpu.VMEM((NCH, CH), jnp.int32),
            pltpu.VMEM((CH, D), jnp.float32),
            pltpu.VMEM((CH, D), jnp.float32),
            pltpu.SemaphoreType.DMA,
            pltpu.SemaphoreType.DMA,
        ],
    )
    return f(idx, codebook)


def kernel(inputs, codebook):
    b, t, d = inputs.shape
    x = inputs.reshape(b * t, d)
    csq = lax.optimization_barrier(
        jnp.sum(codebook * codebook, axis=-1))[None, :]
    xsq = lax.optimization_barrier(
        jnp.sum(x * x, axis=-1, keepdims=True))
    idx = _assign(x, xsq, codebook, csq)
    idx = idx.reshape(NW, NCH, CH)
    quant = _gather(idx, codebook)
    return quant.reshape(b, t, d)


# speedup vs baseline: 1.4364x; 1.0318x over previous
import functools

import jax
import jax.numpy as jnp
from jax import lax
from jax.experimental import pallas as pl
from jax.experimental.pallas import tpu as pltpu
from jax.experimental.pallas import tpu_sc as plsc

BT = 16384
D = 256
K = 8192
MT = 256
CK = 2048
NK = K // CK

NC = 2
NS = 16
NW = NC * NS
ROWS_W = BT // NW
CH = 128
NCH = ROWS_W // CH


PHASES = ((0, 2816), (2816, 5632), (5632, 8192))


def _assign_body(x_ref, xsq_ref, cb_ref, csq_ref, out_ref):
    x = x_ref[...]
    xsq = xsq_ref[...]
    x2_16 = (2.0 * x).astype(jnp.bfloat16)
    lane = lax.broadcasted_iota(jnp.int32, (MT, 128), 1)
    acc_v = None
    acc_i = None
    for (lo, hi) in PHASES:
        cb16 = cb_ref[lo:hi, :]
        cross2 = lax.dot_general(x2_16, cb16,
                                 (((1,), (1,)), ((), ())),
                                 preferred_element_type=jnp.float32)
        pv = None
        pj = jnp.zeros((MT, 128), dtype=jnp.int32)
        for j in range((hi - lo) // 128):
            csq_j = csq_ref[:, lo + j * 128:lo + (j + 1) * 128]
            s = jnp.maximum((xsq + csq_j) - cross2[:, j * 128:(j + 1) * 128], 0.0)
            dj = jnp.where(s == 0.0, 0.0, s * lax.rsqrt(s))
            if pv is None:
                pv = dj
            else:
                better = dj < pv
                pv = jnp.where(better, dj, pv)
                pj = jnp.where(better, j, pj)
        pk = lo + pj * 128 + lane
        lv = jnp.min(pv, axis=1, keepdims=True)
        li = jnp.min(jnp.where(pv == lv, pk, K), axis=1, keepdims=True)
        if acc_v is None:
            acc_v, acc_i = lv, li
        else:
            better = (lv < acc_v) | ((lv == acc_v) & (li < acc_i))
            acc_i = jnp.where(better, li, acc_i)
            acc_v = jnp.where(better, lv, acc_v)
        u = lax.bitcast_convert_type(acc_v, jnp.uint32)
        u = (u + jnp.uint32(0x7FFF) + ((u >> 16) & jnp.uint32(1))) & jnp.uint32(0xFFFF0000)
        acc_v = lax.bitcast_convert_type(u, jnp.float32)
    out_ref[...] = acc_i


def _assign(x, xsq, codebook16, csq):
    return pl.pallas_call(
        _assign_body,
        grid=(BT // MT,),
        in_specs=[
            pl.BlockSpec((MT, D), lambda m: (m, 0)),
            pl.BlockSpec((MT, 1), lambda m: (m, 0)),
            pl.BlockSpec((K, D), lambda m: (0, 0)),
            pl.BlockSpec((1, K), lambda m: (0, 0)),
        ],
        out_specs=pl.BlockSpec((MT, 1), lambda m: (m, 0)),
        out_shape=jax.ShapeDtypeStruct((BT, 1), jnp.int32),
    )(x, xsq, codebook16, csq)


def _gather_body(idx_hbm, cb_hbm, out_hbm, idx_v, buf0, buf1, s0, s1):
    wid = lax.axis_index("s") * NC + lax.axis_index("c")
    base = wid * ROWS_W
    pltpu.sync_copy(idx_hbm.at[wid], idx_v)
    bufs = (buf0, buf1)
    sems = (s0, s1)
    cp = pltpu.async_copy(cb_hbm.at[idx_v.at[0]], bufs[0], sems[0])
    for c in range(NCH):
        cp.wait()
        if c + 1 < NCH:
            cp = pltpu.async_copy(
                cb_hbm.at[idx_v.at[c + 1]], bufs[(c + 1) % 2], sems[(c + 1) % 2])
        pltpu.sync_copy(bufs[c % 2], out_hbm.at[pl.ds(base + c * CH, CH)])


def _gather(idx, codebook):
    mesh = plsc.VectorSubcoreMesh(core_axis_name="c", subcore_axis_name="s")
    f = pl.kernel(
        _gather_body,
        out_type=jax.ShapeDtypeStruct((BT, D), jnp.float32),
        mesh=mesh,
        scratch_types=[
            pltpu.VMEM((NCH, CH), jnp.int32),
            pltpu.VMEM((CH, D), jnp.float32),
            pltpu.VMEM((CH, D), jnp.float32),
            pltpu.SemaphoreType.DMA,
            pltpu.SemaphoreType.DMA,
        ],
    )
    return f(idx, codebook)


def kernel(inputs, codebook):
    b, t, d = inputs.shape
    x = inputs.reshape(b * t, d)
    csq = lax.optimization_barrier(
        jnp.sum(codebook * codebook, axis=-1))[None, :]
    xsq = lax.optimization_barrier(
        jnp.sum(x * x, axis=-1, keepdims=True))
    cb16 = codebook.astype(jnp.bfloat16)
    idx = _assign(x, xsq, cb16, csq)
    idx = idx.reshape(NW, NCH, CH)
    quant = _gather(idx, codebook)
    return quant.reshape(b, t, d)
